# Initial kernel scaffold; baseline (speedup 1.0000x reference)
import jax
import jax.numpy as jnp
from jax.experimental import pallas as pl


def kernel(categorical, numerical, emb_tables, W_num, b_num, W_fus, b_fus):
    raise NotImplementedError("write your pallas kernel here")


# speedup vs baseline: 6.7763x; 6.7763x over previous
import functools

import jax
import jax.numpy as jnp
from jax import lax
from jax.experimental import pallas as pl
from jax.experimental.pallas import tpu as pltpu
from jax.experimental.pallas import tpu_sc as plsc

_B, _L, _NF = 4096, 50, 26
_VOCAB, _EMB = 100000, 32
_NUM, _HID = 16, 128
_T = _B * _L
_CAT = _NF * _EMB
_NC, _NS = 2, 16
_NW = _NC * _NS
_TOK_W = _T // _NW
_NT = 64
_CE = _NT * _NF
_NCHUNK = _TOK_W // _NT


def _sc_gather_body(idx_hbm, offs_hbm, tab_hbm, cat_hbm,
                    raw_v, gidx_v, rows_v, offs_v, sem):
    wid = lax.axis_index("s") * _NC + lax.axis_index("c")
    base = wid * (_TOK_W * _NF)
    pltpu.sync_copy(offs_hbm, offs_v)

    def chunk(i, carry):
        e0 = base + i * _CE
        pltpu.sync_copy(idx_hbm.at[pl.ds(e0, _CE)], raw_v)

        def addk(k, c):
            sl = pl.ds(k * 16, 16)
            gidx_v[sl] = raw_v[sl] + offs_v[sl]
            return c

        lax.fori_loop(0, _CE // 16, addk, 0)
        pltpu.async_copy(tab_hbm.at[gidx_v], rows_v, sem).wait()
        pltpu.sync_copy(rows_v, cat_hbm.at[pl.ds(e0, _CE)])
        return carry

    lax.fori_loop(0, _NCHUNK, chunk, 0)


_sc_gather = functools.partial(
    pl.kernel,
    out_type=jax.ShapeDtypeStruct((_T * _NF, _EMB), jnp.float32),
    mesh=plsc.VectorSubcoreMesh(core_axis_name="c", subcore_axis_name="s"),
    compiler_params=pltpu.CompilerParams(use_tc_tiling_on_sc=False),
    scratch_types=[
        pltpu.VMEM((_CE,), jnp.int32),
        pltpu.VMEM((_CE,), jnp.int32),
        pltpu.VMEM((_CE, _EMB), jnp.float32),
        pltpu.VMEM((_CE,), jnp.int32),
        pltpu.SemaphoreType.DMA,
    ],
)(_sc_gather_body)


def _mm_body(cat_ref, num_ref, wn_ref, bn_ref, wf_ref, bf_ref, out_ref):
    nf = jnp.dot(num_ref[...], wn_ref[...],
                 preferred_element_type=jnp.float32) + bn_ref[...]
    acc = jnp.dot(cat_ref[...], wf_ref[:_CAT, :],
                  preferred_element_type=jnp.float32)
    acc = acc + jnp.dot(nf, wf_ref[_CAT:, :],
                        preferred_element_type=jnp.float32)
    out_ref[...] = acc + bf_ref[...]


_BT = 512


def _tc_matmul(cat, num, wn, bn, wf, bf):
    return pl.pallas_call(
        _mm_body,
        grid=(_T // _BT,),
        in_specs=[
            pl.BlockSpec((_BT, _CAT), lambda i: (i, 0)),
            pl.BlockSpec((_BT, _NUM), lambda i: (i, 0)),
            pl.BlockSpec((_NUM, _HID), lambda i: (0, 0)),
            pl.BlockSpec((1, _HID), lambda i: (0, 0)),
            pl.BlockSpec((_CAT + _HID, _HID), lambda i: (0, 0)),
            pl.BlockSpec((1, _HID), lambda i: (0, 0)),
        ],
        out_specs=pl.BlockSpec((_BT, _HID), lambda i: (i, 0)),
        out_shape=jax.ShapeDtypeStruct((_T, _HID), jnp.float32),
    )(cat, num, wn, bn, wf, bf)


def kernel(categorical, numerical, emb_tables, W_num, b_num, W_fus, b_fus):
    idx_flat = categorical.astype(jnp.int32).reshape(_T * _NF)
    offs = jnp.tile(jnp.arange(_NF, dtype=jnp.int32) * _VOCAB, _NT)
    tab_flat = emb_tables.reshape(_NF * _VOCAB, _EMB)
    cat2 = _sc_gather(idx_flat, offs, tab_flat)
    cat = cat2.reshape(_T, _CAT)
    out = _tc_matmul(cat, numerical.reshape(_T, _NUM), W_num,
                     b_num.reshape(1, _HID), W_fus, b_fus.reshape(1, _HID))
    return out.reshape(_B, _L, _HID)
